# baseline (device time: 47979 ns/iter reference)
import os

import jax
import jax.numpy as jnp
from jax import lax
from jax.experimental import pallas as pl
from jax.experimental.pallas import tpu as pltpu

N_DEV = 4
B = 2
SQ = 512
SKV_SHARD = 512
HQ = 8
DH = 64
HD = HQ * DH
DM = 768
F32 = jnp.float32
BF16 = jnp.bfloat16


def kernel(x, Wq, K_ext, V_ext, Wo):
    def body(x_ref, wq_ref, k_ref, v_ref, wo_ref, out_ref,
             ctx_all, stats_all,
             c_send, c_recv, s_send, s_recv):
        my = lax.axis_index("i")

        barrier = pltpu.get_barrier_semaphore()
        for d in range(1, N_DEV):
            pl.semaphore_signal(barrier, inc=1,
                                device_id=(lax.rem(my + d, N_DEV),),
                                device_id_type=pl.DeviceIdType.MESH)
        pl.semaphore_wait(barrier, N_DEV - 1)

        wq = wq_ref[...].astype(BF16)
        wo = wo_ref[...].astype(BF16)
        ri = lax.broadcasted_iota(jnp.int32, (SQ, SKV_SHARD), 0)
        ci = lax.broadcasted_iota(jnp.int32, (SQ, SKV_SHARD), 1)
        bias = jnp.where((ri // 64) % 4 == (ci // 64) % 4,
                         0.0, -1e9).astype(F32)

        rdmas = [[] for _ in range(B)]

        for b in range(B):
            xb = x_ref[b].astype(BF16)
            q = jnp.dot(xb, wq, preferred_element_type=F32)
            q = (q * 0.125).astype(BF16)
            kb = k_ref[b].astype(BF16)
            vb = v_ref[b].astype(BF16)
            m_rows = []
            l_rows = []
            for h in range(HQ):
                lo, hi = DH * h, DH * (h + 1)
                qh = q[:, lo:hi]
                kh = kb[:, h, :]
                vh = vb[:, h, :]
                s_ = lax.dot_general(
                    qh, kh, (((1,), (1,)), ((), ())),
                    preferred_element_type=F32) + bias
                m = jnp.max(s_, axis=1, keepdims=True)
                p = jnp.exp(s_ - m)
                l = jnp.sum(p, axis=1, keepdims=True)
                ctx_u = jnp.dot(p.astype(BF16), vh,
                                preferred_element_type=F32)
                ctx_all[0, b, :, lo:hi] = ctx_u.astype(BF16)
                m_rows.append(jnp.transpose(m))
                l_rows.append(jnp.transpose(l))
            stats_all[0, b, 0] = jnp.concatenate(m_rows, axis=0)
            stats_all[0, b, 1] = jnp.concatenate(l_rows, axis=0)

            if not os.environ.get("ABLATE_COMM"):
                for d in (1, 2, 3):
                    tgt = lax.rem(my + d, N_DEV)
                    for (buf, ss, rs) in ((ctx_all, c_send, c_recv),
                                          (stats_all, s_send, s_recv)):
                        r = pltpu.make_async_remote_copy(
                            src_ref=buf.at[0, b],
                            dst_ref=buf.at[N_DEV - d, b],
                            send_sem=ss.at[b, d - 1],
                            recv_sem=rs.at[b, d - 1],
                            device_id=(tgt,),
                            device_id_type=pl.DeviceIdType.MESH)
                        r.start()
                        rdmas[b].append(r)

        for b in range(B):
            for r in rdmas[b]:
                r.wait_recv()
            ms = [stats_all[s, b, 0] for s in range(N_DEV)]
            ls = [stats_all[s, b, 1] for s in range(N_DEV)]
            mx = ms[0]
            for m_ in ms[1:]:
                mx = jnp.maximum(mx, m_)
            ws = [jnp.exp(m_ - mx) for m_ in ms]
            ll = ls[0] * ws[0]
            for l_, w_ in zip(ls[1:], ws[1:]):
                ll = ll + l_ * w_
            ws_t = [jnp.transpose(w_) for w_ in ws]
            ll_t = jnp.transpose(ll)
            cs = [ctx_all[s, b] for s in range(N_DEV)]
            heads = []
            for h in range(HQ):
                lo, hi = DH * h, DH * (h + 1)
                acc = cs[0][:, lo:hi].astype(F32) * ws_t[0][:, h:h + 1]
                for s in range(1, N_DEV):
                    acc = acc + cs[s][:, lo:hi].astype(F32) * ws_t[s][:, h:h + 1]
                heads.append((acc / ll_t[:, h:h + 1]).astype(BF16))
            ctx = jnp.concatenate(heads, axis=1)
            out_ref[b] = jnp.dot(ctx, wo, preferred_element_type=F32)

        for b in range(B):
            for r in rdmas[b]:
                r.wait_send()

    return pl.pallas_call(
        body,
        out_shape=jax.ShapeDtypeStruct((B, SQ, DM), F32),
        in_specs=[pl.BlockSpec(memory_space=pltpu.VMEM)] * 5,
        out_specs=pl.BlockSpec(memory_space=pltpu.VMEM),
        scratch_shapes=[
            pltpu.VMEM((N_DEV, B, SQ, HD), BF16),
            pltpu.VMEM((N_DEV, B, 2, HQ, SQ), F32),
            pltpu.SemaphoreType.DMA((B, N_DEV - 1)),
            pltpu.SemaphoreType.DMA((B, N_DEV - 1)),
            pltpu.SemaphoreType.DMA((B, N_DEV - 1)),
            pltpu.SemaphoreType.DMA((B, N_DEV - 1)),
        ],
        compiler_params=pltpu.CompilerParams(collective_id=0),
    )(x, Wq, K_ext, V_ext, Wo)
